# baseline (speedup 1.0000x reference)
import functools

import jax
import jax.numpy as jnp
import numpy as np
from jax import lax
from jax.experimental import pallas as pl
from jax.experimental.pallas import tpu as pltpu
from jax.experimental.pallas import tpu_sc as plsc

_N = 10000
_H = 128
_JB = 2000
_IB = 1024
_GPB = 128

_NBMR = (_N // _JB) * _GPB
_NBMC = -(-_N // _IB) * _IB
_BMW = _NBMR * _NBMC

_NW = 32
_WPT = _BMW // _NW
_NCH = 4
_CHUNK = _WPT // _NCH
_VCAP = 8192
_EBLK = 2048
_ECAP = 82 * _EBLK
_PADR = _N
_SENT = _PADR
_AGGR = _N + 240
_RPT = _AGGR // 16
_EBAT = 128


def _agg_pack_body(ht_ref, hn_ref, s_ref, adj_ref, zt_ref, bm_ref, *, n):
    i = pl.program_id(0)
    j = pl.program_id(1)

    @pl.when(j == 0)
    def _():
        zt_ref[...] = ht_ref[...]

    ab = adj_ref[...].astype(jnp.bfloat16)
    hb = hn_ref[pl.ds(j * _JB, _JB), :].astype(jnp.bfloat16)
    zt_ref[...] += lax.dot_general(
        hb, ab, (((0,), (0,)), ((), ())),
        preferred_element_type=jnp.float32)
    pm = jnp.dot(s_ref[...].astype(jnp.bfloat16), ab,
                 preferred_element_type=jnp.float32)
    colmask = (lax.broadcasted_iota(jnp.int32, (_GPB, _IB), 1)
               + i * _IB) < n
    bm_ref[...] = jnp.where(colmask, pm, 0.0)


def _mlp_core(zt, w1t_ref, b1_ref, g_ref, be_ref, w2t_ref, b2_ref,
              ht_ref, hn_ref, p_ref):
    u = jnp.dot(w1t_ref[...], zt,
                preferred_element_type=jnp.float32) + b1_ref[...]
    mu = jnp.mean(u, axis=1, keepdims=True)
    d = u - mu
    var = jnp.mean(d * d, axis=1, keepdims=True)
    y = g_ref[...] * d * lax.rsqrt(var + 1e-5) + be_ref[...]
    y = jnp.maximum(y, 0.0)
    h = jnp.dot(w2t_ref[...], y,
                preferred_element_type=jnp.float32) + b2_ref[...]
    h = jnp.maximum(h, 0.0)
    ht_ref[...] = h
    hn_ref[...] = h.T
    p_ref[...] = jnp.mean(h, axis=1, keepdims=True)


def _mlp_body(zt_ref, w1t_ref, b1_ref, g_ref, be_ref, w2t_ref, b2_ref,
              ht_ref, hn_ref, p_ref):
    _mlp_core(zt_ref[...], w1t_ref, b1_ref, g_ref, be_ref, w2t_ref, b2_ref,
              ht_ref, hn_ref, p_ref)


def _mlp_sc_body(ht_ref, a_ref, w1t_ref, b1_ref, g_ref, be_ref, w2t_ref,
                 b2_ref, ht_out, hn_out, p_ref, *, n):
    an = a_ref[0, :n, :] + a_ref[1, :n, :]
    zt = ht_ref[...] + an.T
    _mlp_core(zt, w1t_ref, b1_ref, g_ref, be_ref, w2t_ref, b2_ref,
              ht_out, hn_out, p_ref)


def _head_body(p1_ref, p2_ref, p3_ref, w1t_ref, b1_ref, w2t_ref, b2_ref,
               out_ref):
    p = jnp.concatenate([p1_ref[...], p2_ref[...], p3_ref[...]], axis=0)
    t = jnp.dot(w1t_ref[...], p,
                preferred_element_type=jnp.float32) + b1_ref[...]
    t = jnp.maximum(t, 0.0)
    o = jnp.dot(w2t_ref[...], t,
                preferred_element_type=jnp.float32) + b2_ref[...]
    m = jnp.max(o, axis=1, keepdims=True)
    out_ref[...] = o - m - jnp.log(
        jnp.sum(jnp.exp(o - m), axis=1, keepdims=True))


def _iota16():
    return lax.iota(jnp.int32, 16)


def _append(buf, vals, mask, base):
    mi = mask.astype(jnp.int32)
    pos = base + jnp.cumsum(mi) - 1
    plsc.store_scatter(buf, [pos], vals, mask=mask)
    return jnp.sum(mi)


def _extract_body(bm_hbm, edges_hbm, counts_hbm,
                  chunk_v, vbuf, pbuf, ebuf, cvec, cnts, sem):
    w = lax.axis_index("s") * 2 + lax.axis_index("c")
    iota = _iota16()
    cnts[0] = 0
    cnts[1] = 0
    cnts[2] = 0

    def flush():
        nf = cnts[2]
        off = pl.multiple_of(w * _ECAP + nf * _EBLK, _EBLK)
        pltpu.sync_copy(ebuf.at[pl.ds(0, _EBLK)],
                        edges_hbm.at[pl.ds(off, _EBLK)])
        for i in range(16):
            ebuf[pl.ds(i * 16, 16)] = ebuf[pl.ds(_EBLK + i * 16, 16)]
        cnts[1] = cnts[1] - _EBLK
        cnts[2] = nf + 1

    def drain():
        nv = cnts[0]
        ng = (nv + 15) // 16

        def g_body(g, _):
            vals = vbuf[pl.ds(g * 16, 16)].astype(jnp.int32)
            poss = pbuf[pl.ds(g * 16, 16)]
            lane_ok = iota < (nv - g * 16)
            gw = w * _WPT + poss
            r = gw // _NBMC
            c = gw - r * _NBMC
            jj = r // _GPB
            q = r - jj * _GPB
            sb = jj * _JB + q * 16
            base = lax.shift_left(sb, 14) | c
            ne = cnts[1]
            for t in range(16):
                mt = jnp.logical_and(
                    (lax.shift_right_logical(vals, t) & 1) != 0, lane_ok)
                ne = ne + _append(ebuf, base + (t << 14), mt, ne)
            cnts[1] = ne

            @pl.when(ne >= _EBLK)
            def _():
                flush()

            return 0

        lax.fori_loop(0, ng, g_body, 0)
        cnts[0] = 0

    for ch in range(_NCH):
        pltpu.sync_copy(
            bm_hbm.at[pl.ds(pl.multiple_of(w * _WPT + ch * _CHUNK, _CHUNK),
                            _CHUNK)], chunk_v)

        def scan_body(k, _, ch=ch):
            v = chunk_v[pl.ds(k * 16, 16)]
            m = v != 0.0
            nv0 = cnts[0]
            nz = _append(vbuf, v, m, nv0)
            _append(pbuf, (ch * _CHUNK + k * 16) + iota, m, nv0)
            cnts[0] = nv0 + nz

            @pl.when(cnts[0] >= _VCAP - 16)
            def _():
                drain()

            return 0

        lax.fori_loop(0, _CHUNK // 16, scan_body, 0)

    drain()
    ne = cnts[1]

    def pad_body(i, _):
        cur = ebuf[pl.ds(i * 16, 16)]
        ebuf[pl.ds(i * 16, 16)] = jnp.where(i * 16 + iota >= ne, _SENT, cur)
        return 0

    lax.fori_loop(0, _EBLK // 16, pad_body, 0)

    @pl.when(ne > 0)
    def _():
        off = pl.multiple_of(w * _ECAP + cnts[2] * _EBLK, _EBLK)
        pltpu.sync_copy(ebuf.at[pl.ds(0, _EBLK)],
                        edges_hbm.at[pl.ds(off, _EBLK)])

    cvec[...] = jnp.full((16,), cnts[2] * _EBLK + ne, jnp.int32)
    pltpu.sync_copy(cvec,
                    counts_hbm.at[pl.ds(pl.multiple_of(w * 16, 16), 16)])


def _segsum_body(h_hbm, z0_hbm, edges_hbm, counts_hbm, agg_hbm,
                 aggsh, ebat, sidx, didx, rows, cvec, sem):
    cid = lax.axis_index("c")
    sid = lax.axis_index("s")
    w = sid * 2 + cid
    row0 = pl.multiple_of(sid * _RPT, _RPT)
    pltpu.sync_copy(z0_hbm.at[pl.ds(row0, _RPT)],
                    aggsh.at[pl.ds(row0, _RPT)])
    plsc.subcore_barrier()
    pltpu.sync_copy(counts_hbm.at[pl.ds(pl.multiple_of(w * 16, 16), 16)],
                    cvec)
    cnt = jnp.sum(jnp.where(_iota16() == 0, cvec[...], 0))
    nb = (cnt + _EBAT - 1) // _EBAT

    def b_body(b, _):
        eoff = pl.multiple_of(w * _ECAP + b * _EBAT, _EBAT)
        pltpu.sync_copy(edges_hbm.at[pl.ds(eoff, _EBAT)], ebat)
        for v in range(_EBAT // 16):
            e = ebat[pl.ds(v * 16, 16)]
            sidx[pl.ds(v * 16, 16)] = lax.shift_right_logical(e, 14)
            didx[pl.ds(v * 16, 16)] = e & 16383
        pltpu.async_copy(h_hbm.at[sidx], rows, sem).wait()
        pltpu.sync_copy(rows, aggsh.at[didx], add=True)
        return 0

    lax.fori_loop(0, nb, b_body, 0)
    plsc.subcore_barrier()
    pltpu.sync_copy(aggsh.at[pl.ds(row0, _RPT)],
                    agg_hbm.at[cid, pl.ds(row0, _RPT)])


def _pack_matrix():
    rr = np.arange(_JB)
    s = np.zeros((_GPB, _JB), np.float32)
    s[rr // 16, rr] = (2.0 ** (rr % 16)).astype(np.float32)
    return s


def kernel(features, adj, c1_W1, c1_b1, c1_g, c1_be, c1_W2, c1_b2,
           c2_W1, c2_b1, c2_g, c2_be, c2_W2, c2_b2,
           c3_W1, c3_b1, c3_g, c3_be, c3_W2, c3_b2,
           m_W1, m_b1, m_W2, m_b2):
    n, dim = features.shape
    h = c1_W1.shape[1]
    jb, ib = _JB, _IB
    grid = (pl.cdiv(n, ib), n // jb)

    agg_pack = pl.pallas_call(
        functools.partial(_agg_pack_body, n=n),
        grid=grid,
        in_specs=[
            pl.BlockSpec((dim, ib), lambda i, j: (0, i)),
            pl.BlockSpec((n, dim), lambda i, j: (0, 0)),
            pl.BlockSpec((_GPB, jb), lambda i, j: (0, 0)),
            pl.BlockSpec((jb, ib), lambda i, j: (j, i)),
        ],
        out_specs=(
            pl.BlockSpec((dim, ib), lambda i, j: (0, i)),
            pl.BlockSpec((_GPB, ib), lambda i, j: (j, i)),
        ),
        out_shape=(
            jax.ShapeDtypeStruct((dim, n), jnp.float32),
            jax.ShapeDtypeStruct((_NBMR, _NBMC), jnp.float32),
        ),
    )

    mlp = pl.pallas_call(
        _mlp_body,
        out_shape=(
            jax.ShapeDtypeStruct((h, n), jnp.float32),
            jax.ShapeDtypeStruct((n, h), jnp.float32),
            jax.ShapeDtypeStruct((h, 1), jnp.float32),
        ),
    )

    mlp_sc = pl.pallas_call(
        functools.partial(_mlp_sc_body, n=n),
        out_shape=(
            jax.ShapeDtypeStruct((h, n), jnp.float32),
            jax.ShapeDtypeStruct((n, h), jnp.float32),
            jax.ShapeDtypeStruct((h, 1), jnp.float32),
        ),
    )

    head = pl.pallas_call(
        _head_body,
        out_shape=jax.ShapeDtypeStruct((1, 1), jnp.float32),
    )

    mesh = plsc.VectorSubcoreMesh(core_axis_name="c", subcore_axis_name="s")

    extract = functools.partial(
        pl.kernel, _extract_body, mesh=mesh,
        out_type=(
            jax.ShapeDtypeStruct((_NW * _ECAP,), jnp.int32),
            jax.ShapeDtypeStruct((_NW * 16,), jnp.int32),
        ),
        compiler_params=pltpu.CompilerParams(needs_layout_passes=False),
        scratch_types=[
            pltpu.VMEM((_CHUNK,), jnp.float32),
            pltpu.VMEM((_VCAP,), jnp.float32),
            pltpu.VMEM((_VCAP,), jnp.int32),
            pltpu.VMEM((_EBLK + 256,), jnp.int32),
            pltpu.VMEM((16,), jnp.int32),
            pltpu.SMEM((8,), jnp.int32),
            pltpu.SemaphoreType.DMA,
        ],
    )()

    segsum = functools.partial(
        pl.kernel, _segsum_body, mesh=mesh,
        out_type=jax.ShapeDtypeStruct((2, _AGGR, h), jnp.float32),
        compiler_params=pltpu.CompilerParams(needs_layout_passes=False),
        scratch_types=[
            pltpu.VMEM_SHARED((_AGGR, h), jnp.float32),
            pltpu.VMEM((_EBAT,), jnp.int32),
            pltpu.VMEM((_EBAT,), jnp.int32),
            pltpu.VMEM((_EBAT,), jnp.int32),
            pltpu.VMEM((_EBAT, h), jnp.float32),
            pltpu.VMEM((16,), jnp.int32),
            pltpu.SemaphoreType.DMA,
        ],
    )()

    def mlp_call(fn, first, W1, b1, g, be, W2, b2):
        return fn(*first, W1.T, b1[:, None], g[:, None], be[:, None],
                  W2.T, b2[:, None])

    s_mat = jnp.asarray(_pack_matrix())
    ht0 = features.T
    z1t, bm = agg_pack(ht0, features, s_mat, adj)
    h1t, h1n, p1 = mlp_call(mlp, (z1t,), c1_W1, c1_b1, c1_g, c1_be,
                            c1_W2, c1_b2)
    edges, counts = extract(bm.reshape(-1))
    z0 = jnp.zeros((_AGGR, h), jnp.float32)
    a2 = segsum(h1n, z0, edges, counts)
    h2t, h2n, p2 = mlp_call(mlp_sc, (h1t, a2), c2_W1, c2_b1, c2_g, c2_be,
                            c2_W2, c2_b2)
    a3 = segsum(h2n, z0, edges, counts)
    _, _, p3 = mlp_call(mlp_sc, (h2t, a3), c3_W1, c3_b1, c3_g, c3_be,
                        c3_W2, c3_b2)
    out = head(p1, p2, p3, m_W1.T, m_b1[:, None], m_W2.T, m_b2[:, None])
    return out[0]


# speedup vs baseline: 2.3290x; 2.3290x over previous
import functools

import jax
import jax.numpy as jnp
from jax.experimental import pallas as pl

_JB = 2000
_IB = 1024


def _agg_cast_body(ht_ref, hn_ref, adj_ref, zt_ref, adjbf_ref):
    j = pl.program_id(1)

    @pl.when(j == 0)
    def _():
        zt_ref[...] = ht_ref[...]

    hb = hn_ref[pl.ds(j * _JB, _JB), :].astype(jnp.bfloat16)
    ab = adj_ref[...].astype(jnp.bfloat16)
    adjbf_ref[...] = ab
    zt_ref[...] += jax.lax.dot_general(
        hb, ab, (((0,), (0,)), ((), ())),
        preferred_element_type=jnp.float32)


def _agg_bf_body(ht_ref, hn_ref, adjbf_ref, zt_ref):
    j = pl.program_id(1)

    @pl.when(j == 0)
    def _():
        zt_ref[...] = ht_ref[...]

    hb = hn_ref[pl.ds(j * _JB, _JB), :].astype(jnp.bfloat16)
    zt_ref[...] += jax.lax.dot_general(
        hb, adjbf_ref[...], (((0,), (0,)), ((), ())),
        preferred_element_type=jnp.float32)


def _mlp_body(zt_ref, w1t_ref, b1_ref, g_ref, be_ref, w2t_ref, b2_ref,
              ht_ref, hn_ref, p_ref):
    z = zt_ref[...]
    u = jnp.dot(w1t_ref[...], z,
                preferred_element_type=jnp.float32) + b1_ref[...]
    mu = jnp.mean(u, axis=1, keepdims=True)
    d = u - mu
    var = jnp.mean(d * d, axis=1, keepdims=True)
    y = g_ref[...] * d * jax.lax.rsqrt(var + 1e-5) + be_ref[...]
    y = jnp.maximum(y, 0.0)
    h = jnp.dot(w2t_ref[...], y,
                preferred_element_type=jnp.float32) + b2_ref[...]
    h = jnp.maximum(h, 0.0)
    ht_ref[...] = h
    hn_ref[...] = h.T
    p_ref[...] = jnp.mean(h, axis=1, keepdims=True)


def _head_body(p1_ref, p2_ref, p3_ref, w1t_ref, b1_ref, w2t_ref, b2_ref,
               out_ref):
    p = jnp.concatenate([p1_ref[...], p2_ref[...], p3_ref[...]], axis=0)
    t = jnp.dot(w1t_ref[...], p,
                preferred_element_type=jnp.float32) + b1_ref[...]
    t = jnp.maximum(t, 0.0)
    o = jnp.dot(w2t_ref[...], t,
                preferred_element_type=jnp.float32) + b2_ref[...]
    m = jnp.max(o, axis=1, keepdims=True)
    out_ref[...] = o - m - jnp.log(
        jnp.sum(jnp.exp(o - m), axis=1, keepdims=True))


def kernel(features, adj, c1_W1, c1_b1, c1_g, c1_be, c1_W2, c1_b2,
           c2_W1, c2_b1, c2_g, c2_be, c2_W2, c2_b2,
           c3_W1, c3_b1, c3_g, c3_be, c3_W2, c3_b2,
           m_W1, m_b1, m_W2, m_b2):
    n, dim = features.shape
    h = c1_W1.shape[1]
    jb, ib = _JB, _IB
    assert n % jb == 0
    grid = (pl.cdiv(n, ib), n // jb)

    agg_cast = pl.pallas_call(
        _agg_cast_body,
        grid=grid,
        in_specs=[
            pl.BlockSpec((dim, ib), lambda i, j: (0, i)),
            pl.BlockSpec((n, dim), lambda i, j: (0, 0)),
            pl.BlockSpec((jb, ib), lambda i, j: (j, i)),
        ],
        out_specs=(
            pl.BlockSpec((dim, ib), lambda i, j: (0, i)),
            pl.BlockSpec((jb, ib), lambda i, j: (j, i)),
        ),
        out_shape=(
            jax.ShapeDtypeStruct((dim, n), jnp.float32),
            jax.ShapeDtypeStruct((n, n), jnp.bfloat16),
        ),
    )

    agg_bf = pl.pallas_call(
        _agg_bf_body,
        grid=grid,
        in_specs=[
            pl.BlockSpec((dim, ib), lambda i, j: (0, i)),
            pl.BlockSpec((n, dim), lambda i, j: (0, 0)),
            pl.BlockSpec((jb, ib), lambda i, j: (j, i)),
        ],
        out_specs=pl.BlockSpec((dim, ib), lambda i, j: (0, i)),
        out_shape=jax.ShapeDtypeStruct((dim, n), jnp.float32),
    )

    mlp = pl.pallas_call(
        _mlp_body,
        out_shape=(
            jax.ShapeDtypeStruct((h, n), jnp.float32),
            jax.ShapeDtypeStruct((n, h), jnp.float32),
            jax.ShapeDtypeStruct((h, 1), jnp.float32),
        ),
    )

    head = pl.pallas_call(
        _head_body,
        out_shape=jax.ShapeDtypeStruct((1, 1), jnp.float32),
    )

    def mlp_call(zt, W1, b1, g, be, W2, b2):
        return mlp(zt, W1.T, b1[:, None], g[:, None], be[:, None],
                   W2.T, b2[:, None])

    ht0 = features.T
    z1t, adj_bf = agg_cast(ht0, features, adj)
    h1t, h1n, p1 = mlp_call(z1t, c1_W1, c1_b1, c1_g, c1_be, c1_W2, c1_b2)
    z2t = agg_bf(h1t, h1n, adj_bf)
    h2t, h2n, p2 = mlp_call(z2t, c2_W1, c2_b1, c2_g, c2_be, c2_W2, c2_b2)
    z3t = agg_bf(h2t, h2n, adj_bf)
    _, _, p3 = mlp_call(z3t, c3_W1, c3_b1, c3_g, c3_be, c3_W2, c3_b2)
    out = head(p1, p2, p3, m_W1.T, m_b1[:, None], m_W2.T, m_b2[:, None])
    return out[0]


# speedup vs baseline: 2.4987x; 1.0729x over previous
import functools

import jax
import jax.numpy as jnp
from jax.experimental import pallas as pl

_JB = 2000
_IB = 1024


def _agg_cast_body(ht_ref, hn_ref, adj_ref, zt_ref, adjbf_ref):
    j = pl.program_id(1)

    @pl.when(j == 0)
    def _():
        zt_ref[...] = ht_ref[...]

    hb = hn_ref[pl.ds(j * _JB, _JB), :].astype(jnp.bfloat16)
    ab = adj_ref[...].astype(jnp.bfloat16)
    adjbf_ref[...] = ab
    zt_ref[...] += jax.lax.dot_general(
        hb, ab, (((0,), (0,)), ((), ())),
        preferred_element_type=jnp.float32)


def _agg_bf_body(ht_ref, hn_ref, adjbf_ref, zt_ref):
    hb = hn_ref[...].astype(jnp.bfloat16)
    zt_ref[...] = ht_ref[...] + jax.lax.dot_general(
        hb, adjbf_ref[...], (((0,), (0,)), ((), ())),
        preferred_element_type=jnp.float32)


def _mlp_body(zt_ref, w1t_ref, b1_ref, g_ref, be_ref, w2t_ref, b2_ref,
              ht_ref, hn_ref, p_ref):
    z = zt_ref[...]
    u = jnp.dot(w1t_ref[...], z,
                preferred_element_type=jnp.float32) + b1_ref[...]
    mu = jnp.mean(u, axis=1, keepdims=True)
    d = u - mu
    var = jnp.mean(d * d, axis=1, keepdims=True)
    y = g_ref[...] * d * jax.lax.rsqrt(var + 1e-5) + be_ref[...]
    y = jnp.maximum(y, 0.0)
    h = jnp.dot(w2t_ref[...], y,
                preferred_element_type=jnp.float32) + b2_ref[...]
    h = jnp.maximum(h, 0.0)
    ht_ref[...] = h
    hn_ref[...] = h.T
    p_ref[...] = jnp.mean(h, axis=1, keepdims=True)


def _head_body(p1_ref, p2_ref, p3_ref, w1t_ref, b1_ref, w2t_ref, b2_ref,
               out_ref):
    p = jnp.concatenate([p1_ref[...], p2_ref[...], p3_ref[...]], axis=0)
    t = jnp.dot(w1t_ref[...], p,
                preferred_element_type=jnp.float32) + b1_ref[...]
    t = jnp.maximum(t, 0.0)
    o = jnp.dot(w2t_ref[...], t,
                preferred_element_type=jnp.float32) + b2_ref[...]
    m = jnp.max(o, axis=1, keepdims=True)
    out_ref[...] = o - m - jnp.log(
        jnp.sum(jnp.exp(o - m), axis=1, keepdims=True))


def kernel(features, adj, c1_W1, c1_b1, c1_g, c1_be, c1_W2, c1_b2,
           c2_W1, c2_b1, c2_g, c2_be, c2_W2, c2_b2,
           c3_W1, c3_b1, c3_g, c3_be, c3_W2, c3_b2,
           m_W1, m_b1, m_W2, m_b2):
    n, dim = features.shape
    h = c1_W1.shape[1]
    jb, ib = _JB, _IB
    assert n % jb == 0
    grid = (pl.cdiv(n, ib), n // jb)

    agg_cast = pl.pallas_call(
        _agg_cast_body,
        grid=grid,
        in_specs=[
            pl.BlockSpec((dim, ib), lambda i, j: (0, i)),
            pl.BlockSpec((n, dim), lambda i, j: (0, 0)),
            pl.BlockSpec((jb, ib), lambda i, j: (j, i)),
        ],
        out_specs=(
            pl.BlockSpec((dim, ib), lambda i, j: (0, i)),
            pl.BlockSpec((jb, ib), lambda i, j: (j, i)),
        ),
        out_shape=(
            jax.ShapeDtypeStruct((dim, n), jnp.float32),
            jax.ShapeDtypeStruct((n, n), jnp.bfloat16),
        ),
    )

    agg_bf = pl.pallas_call(
        _agg_bf_body,
        grid=(pl.cdiv(n, ib),),
        in_specs=[
            pl.BlockSpec((dim, ib), lambda i: (0, i)),
            pl.BlockSpec((n, dim), lambda i: (0, 0)),
            pl.BlockSpec((n, ib), lambda i: (0, i)),
        ],
        out_specs=pl.BlockSpec((dim, ib), lambda i: (0, i)),
        out_shape=jax.ShapeDtypeStruct((dim, n), jnp.float32),
    )

    mlp = pl.pallas_call(
        _mlp_body,
        out_shape=(
            jax.ShapeDtypeStruct((h, n), jnp.float32),
            jax.ShapeDtypeStruct((n, h), jnp.float32),
            jax.ShapeDtypeStruct((h, 1), jnp.float32),
        ),
    )

    head = pl.pallas_call(
        _head_body,
        out_shape=jax.ShapeDtypeStruct((1, 1), jnp.float32),
    )

    def mlp_call(zt, W1, b1, g, be, W2, b2):
        return mlp(zt, W1.T, b1[:, None], g[:, None], be[:, None],
                   W2.T, b2[:, None])

    ht0 = features.T
    z1t, adj_bf = agg_cast(ht0, features, adj)
    h1t, h1n, p1 = mlp_call(z1t, c1_W1, c1_b1, c1_g, c1_be, c1_W2, c1_b2)
    z2t = agg_bf(h1t, h1n, adj_bf)
    h2t, h2n, p2 = mlp_call(z2t, c2_W1, c2_b1, c2_g, c2_be, c2_W2, c2_b2)
    z3t = agg_bf(h2t, h2n, adj_bf)
    _, _, p3 = mlp_call(z3t, c3_W1, c3_b1, c3_g, c3_be, c3_W2, c3_b2)
    out = head(p1, p2, p3, m_W1.T, m_b1[:, None], m_W2.T, m_b2[:, None])
    return out[0]


# speedup vs baseline: 2.5209x; 1.0089x over previous
import functools

import jax
import jax.numpy as jnp
from jax.experimental import pallas as pl

_JB = 2000
_IB = 1024


def _agg_cast_body(ht_ref, hn_ref, adj_ref, zt_ref, adjbf_ref):
    hb = hn_ref[...].astype(jnp.bfloat16)
    ab = adj_ref[...].astype(jnp.bfloat16)
    adjbf_ref[...] = ab
    zt_ref[...] = ht_ref[...] + jax.lax.dot_general(
        hb, ab, (((0,), (0,)), ((), ())),
        preferred_element_type=jnp.float32)


def _agg_bf_body(ht_ref, hn_ref, adjbf_ref, zt_ref):
    hb = hn_ref[...].astype(jnp.bfloat16)
    zt_ref[...] = ht_ref[...] + jax.lax.dot_general(
        hb, adjbf_ref[...], (((0,), (0,)), ((), ())),
        preferred_element_type=jnp.float32)


def _mlp_body(zt_ref, w1t_ref, b1_ref, g_ref, be_ref, w2t_ref, b2_ref,
              ht_ref, hn_ref, p_ref):
    z = zt_ref[...]
    u = jnp.dot(w1t_ref[...], z,
                preferred_element_type=jnp.float32) + b1_ref[...]
    mu = jnp.mean(u, axis=1, keepdims=True)
    d = u - mu
    var = jnp.mean(d * d, axis=1, keepdims=True)
    y = g_ref[...] * d * jax.lax.rsqrt(var + 1e-5) + be_ref[...]
    y = jnp.maximum(y, 0.0)
    h = jnp.dot(w2t_ref[...], y,
                preferred_element_type=jnp.float32) + b2_ref[...]
    h = jnp.maximum(h, 0.0)
    ht_ref[...] = h
    hn_ref[...] = h.T
    p_ref[...] = jnp.mean(h, axis=1, keepdims=True)


def _head_body(p1_ref, p2_ref, p3_ref, w1t_ref, b1_ref, w2t_ref, b2_ref,
               out_ref):
    p = jnp.concatenate([p1_ref[...], p2_ref[...], p3_ref[...]], axis=0)
    t = jnp.dot(w1t_ref[...], p,
                preferred_element_type=jnp.float32) + b1_ref[...]
    t = jnp.maximum(t, 0.0)
    o = jnp.dot(w2t_ref[...], t,
                preferred_element_type=jnp.float32) + b2_ref[...]
    m = jnp.max(o, axis=1, keepdims=True)
    out_ref[...] = o - m - jnp.log(
        jnp.sum(jnp.exp(o - m), axis=1, keepdims=True))


def kernel(features, adj, c1_W1, c1_b1, c1_g, c1_be, c1_W2, c1_b2,
           c2_W1, c2_b1, c2_g, c2_be, c2_W2, c2_b2,
           c3_W1, c3_b1, c3_g, c3_be, c3_W2, c3_b2,
           m_W1, m_b1, m_W2, m_b2):
    n, dim = features.shape
    h = c1_W1.shape[1]
    ib = _IB

    ib1 = 256
    agg_cast = pl.pallas_call(
        _agg_cast_body,
        grid=(pl.cdiv(n, ib1),),
        in_specs=[
            pl.BlockSpec((dim, ib1), lambda i: (0, i)),
            pl.BlockSpec((n, dim), lambda i: (0, 0)),
            pl.BlockSpec((n, ib1), lambda i: (0, i)),
        ],
        out_specs=(
            pl.BlockSpec((dim, ib1), lambda i: (0, i)),
            pl.BlockSpec((n, ib1), lambda i: (0, i)),
        ),
        out_shape=(
            jax.ShapeDtypeStruct((dim, n), jnp.float32),
            jax.ShapeDtypeStruct((n, n), jnp.bfloat16),
        ),
    )

    agg_bf = pl.pallas_call(
        _agg_bf_body,
        grid=(pl.cdiv(n, ib),),
        in_specs=[
            pl.BlockSpec((dim, ib), lambda i: (0, i)),
            pl.BlockSpec((n, dim), lambda i: (0, 0)),
            pl.BlockSpec((n, ib), lambda i: (0, i)),
        ],
        out_specs=pl.BlockSpec((dim, ib), lambda i: (0, i)),
        out_shape=jax.ShapeDtypeStruct((dim, n), jnp.float32),
    )

    mlp = pl.pallas_call(
        _mlp_body,
        out_shape=(
            jax.ShapeDtypeStruct((h, n), jnp.float32),
            jax.ShapeDtypeStruct((n, h), jnp.float32),
            jax.ShapeDtypeStruct((h, 1), jnp.float32),
        ),
    )

    head = pl.pallas_call(
        _head_body,
        out_shape=jax.ShapeDtypeStruct((1, 1), jnp.float32),
    )

    def mlp_call(zt, W1, b1, g, be, W2, b2):
        return mlp(zt, W1.T, b1[:, None], g[:, None], be[:, None],
                   W2.T, b2[:, None])

    ht0 = features.T
    z1t, adj_bf = agg_cast(ht0, features, adj)
    h1t, h1n, p1 = mlp_call(z1t, c1_W1, c1_b1, c1_g, c1_be, c1_W2, c1_b2)
    z2t = agg_bf(h1t, h1n, adj_bf)
    h2t, h2n, p2 = mlp_call(z2t, c2_W1, c2_b1, c2_g, c2_be, c2_W2, c2_b2)
    z3t = agg_bf(h2t, h2n, adj_bf)
    _, _, p3 = mlp_call(z3t, c3_W1, c3_b1, c3_g, c3_be, c3_W2, c3_b2)
    out = head(p1, p2, p3, m_W1.T, m_b1[:, None], m_W2.T, m_b2[:, None])
    return out[0]
